# baseline (speedup 1.0000x reference)
import functools

import jax
import jax.numpy as jnp
from jax import lax
from jax.experimental import pallas as pl
from jax.experimental.pallas import tpu as pltpu
from jax.experimental.pallas import tpu_sc as plsc

_B, _L, _D = 4096, 200, 64
_N = _B * _L
_NC, _NS = 2, 16
_NW = _NC * _NS
_PER_W = _N // _NW
_CHUNK = _L
_NCHUNK = _PER_W // _CHUNK
_IA, _IB = 128, _CHUNK - 128


def _body(x_hbm, tok_hbm, pos_hbm, out_hbm, idx_a, idx_b, rows_v, pos_v, sem):
    wid = lax.axis_index("s") * _NC + lax.axis_index("c")
    base = wid * _PER_W
    pltpu.sync_copy(pos_hbm.at[pl.ds(0, _L)], pos_v)

    def chunk_body(i, carry):
        cbase = base + i * _CHUNK
        pltpu.sync_copy(x_hbm.at[pl.ds(cbase, _IA)], idx_a)
        pltpu.sync_copy(x_hbm.at[pl.ds(cbase + _IA, _IB)], idx_b)
        ca = pltpu.async_copy(tok_hbm.at[idx_a], rows_v.at[pl.ds(0, _IA)], sem)
        cb = pltpu.async_copy(tok_hbm.at[idx_b], rows_v.at[pl.ds(_IA, _IB)], sem)
        ca.wait()
        cb.wait()

        def row_body(r, c2):
            for c in range(_D // 16):
                s = pl.ds(c * 16, 16)
                rows_v[r, s] = rows_v[r, s] + pos_v[r, s]
            return c2

        lax.fori_loop(0, _CHUNK, row_body, 0)
        pltpu.sync_copy(rows_v, out_hbm.at[pl.ds(cbase, _CHUNK)])
        return carry

    lax.fori_loop(0, _NCHUNK, chunk_body, 0)


_sc_embed = functools.partial(
    pl.kernel,
    mesh=plsc.VectorSubcoreMesh(core_axis_name="c", subcore_axis_name="s"),
    out_type=jax.ShapeDtypeStruct((_N, _D), jnp.float32),
    compiler_params=pltpu.CompilerParams(use_tc_tiling_on_sc=False),
    scratch_types=[
        pltpu.VMEM((_IA,), jnp.int32),
        pltpu.VMEM((_IB,), jnp.int32),
        pltpu.VMEM((_CHUNK, _D), jnp.float32),
        pltpu.VMEM((_L, _D), jnp.float32),
        pltpu.SemaphoreType.DMA,
    ],
)(_body)


@jax.jit
def kernel(x, token_table, pos_table):
    out = _sc_embed(x.reshape(-1), token_table, pos_table)
    return out.reshape(_B, _L, _D)


# speedup vs baseline: 1.2452x; 1.2452x over previous
import functools

import jax
import jax.numpy as jnp
from jax import lax
from jax.experimental import pallas as pl
from jax.experimental.pallas import tpu as pltpu
from jax.experimental.pallas import tpu_sc as plsc

_B, _L, _D = 4096, 200, 64
_N = _B * _L
_NC, _NS = 2, 16
_NW = _NC * _NS
_PER_W = _N // _NW
_CHUNK = _L
_NCHUNK = _PER_W // _CHUNK
_IA, _IB = 128, _CHUNK - 128
_NBUF = 4
_AHEAD = 2


def _body(x_hbm, tok_hbm, pos_hbm, out_hbm, idx_all, r0, r1, r2, r3, pos_v,
          g0, g1, g2, g3, s0, s1, s2, s3):
    rows = (r0, r1, r2, r3)
    sem_g = (g0, g1, g2, g3)
    sem_st = (s0, s1, s2, s3)
    wid = lax.axis_index("s") * _NC + lax.axis_index("c")
    base = wid * _PER_W
    pltpu.sync_copy(pos_hbm.at[pl.ds(0, _L)], pos_v)
    pltpu.sync_copy(x_hbm.at[pl.ds(base, _PER_W)], idx_all)

    def fire_gather(b, f):
        coff = f * _CHUNK
        pltpu.async_copy(tok_hbm.at[idx_all.at[pl.ds(coff, _IA)]],
                         rows[b].at[pl.ds(0, _IA)], sem_g[b])
        pltpu.async_copy(tok_hbm.at[idx_all.at[pl.ds(coff + _IA, _IB)]],
                         rows[b].at[pl.ds(_IA, _IB)], sem_g[b])

    def wait_store(b):
        pltpu.make_async_copy(
            rows[b], out_hbm.at[pl.ds(base, _CHUNK)], sem_st[b]).wait()

    def process(b, j):
        buf = rows[b]
        pltpu.make_async_copy(tok_hbm.at[idx_all.at[pl.ds(0, _IA)]],
                              buf.at[pl.ds(0, _IA)], sem_g[b]).wait()
        pltpu.make_async_copy(tok_hbm.at[idx_all.at[pl.ds(0, _IB)]],
                              buf.at[pl.ds(_IA, _IB)], sem_g[b]).wait()

        @plsc.parallel_loop(0, _CHUNK, step=1, unroll=8)
        def _add(r):
            for c in range(_D // 16):
                s = pl.ds(c * 16, 16)
                buf[r, s] = buf[r, s] + pos_v[r, s]

        pltpu.async_copy(
            buf, out_hbm.at[pl.ds(base + j * _CHUNK, _CHUNK)], sem_st[b])

    fire_gather(0, 0)
    fire_gather(1, 1)
    fire_gather(2, 2)
    process(0, 0)
    fire_gather(3, 3)
    process(1, 1)
    wait_store(0)
    fire_gather(0, 4)
    process(2, 2)
    wait_store(1)
    fire_gather(1, 5)
    process(3, 3)

    @pl.loop(4, _NCHUNK - 4, step=_NBUF)
    def _grp(g):
        for b in range(_NBUF):
            fb = (b + _AHEAD) % _NBUF
            wait_store(fb)
            fire_gather(fb, g + b + _AHEAD)
            process(b, g + b)

    wait_store(2)
    fire_gather(2, _NCHUNK - 2)
    wait_store(3)
    fire_gather(3, _NCHUNK - 1)
    process(0, _NCHUNK - 4)
    process(1, _NCHUNK - 3)
    process(2, _NCHUNK - 2)
    process(3, _NCHUNK - 1)
    for b in range(_NBUF):
        wait_store(b)


_sc_embed = functools.partial(
    pl.kernel,
    mesh=plsc.VectorSubcoreMesh(core_axis_name="c", subcore_axis_name="s"),
    out_type=jax.ShapeDtypeStruct((_N, _D), jnp.float32),
    compiler_params=pltpu.CompilerParams(use_tc_tiling_on_sc=False),
    scratch_types=(
        [pltpu.VMEM((_PER_W,), jnp.int32)]
        + [pltpu.VMEM((_CHUNK, _D), jnp.float32) for _ in range(_NBUF)]
        + [pltpu.VMEM((_L, _D), jnp.float32)]
        + [pltpu.SemaphoreType.DMA for _ in range(2 * _NBUF)]
    ),
)(_body)


@jax.jit
def kernel(x, token_table, pos_table):
    out = _sc_embed(x.reshape(-1), token_table, pos_table)
    return out.reshape(_B, _L, _D)
